# Initial kernel scaffold; baseline (speedup 1.0000x reference)
import jax
import jax.numpy as jnp
from jax.experimental import pallas as pl


def kernel(token_ids, embedding_mat):
    raise NotImplementedError("write your pallas kernel here")


# speedup vs baseline: 1.8073x; 1.8073x over previous
import jax
import jax.numpy as jnp
from jax import lax
from jax.experimental import pallas as pl
from jax.experimental.pallas import tpu as pltpu
from jax.experimental.pallas import tpu_sc as plsc

NUM_CORES = 2
NUM_SUBCORES = 16
NUM_WORKERS = NUM_CORES * NUM_SUBCORES

B_TOTAL = 16384 * 50
PER_WORKER = B_TOTAL // NUM_WORKERS
CHUNK = 512
NCHUNK = PER_WORKER // CHUNK
DIM = 64


def _make_kernel():
    mesh = plsc.VectorSubcoreMesh(
        core_axis_name="c", subcore_axis_name="s")

    @pl.kernel(
        out_type=jax.ShapeDtypeStruct((B_TOTAL, DIM), jnp.float32),
        mesh=mesh,
        scratch_types=[
            pltpu.VMEM((CHUNK,), jnp.int32),
            pltpu.VMEM((CHUNK, DIM), jnp.float32),
            pltpu.SemaphoreType.DMA,
        ],
        compiler_params=pltpu.CompilerParams(use_tc_tiling_on_sc=False),
    )
    def emb_kernel(table_hbm, idx_hbm, out_hbm, idx_v, rows_v, sem):
        wid = lax.axis_index("s") * NUM_CORES + lax.axis_index("c")
        wbase = wid * PER_WORKER

        def body(g, carry):
            base = wbase + g * CHUNK
            pltpu.sync_copy(idx_hbm.at[pl.ds(base, CHUNK)], idx_v)
            pltpu.async_copy(table_hbm.at[idx_v], rows_v, sem).wait()
            pltpu.sync_copy(rows_v, out_hbm.at[pl.ds(base, CHUNK)])
            return carry

        lax.fori_loop(0, NCHUNK, body, 0)

    return emb_kernel


_emb_kernel = _make_kernel()


@jax.jit
def kernel(token_ids, embedding_mat):
    shape = token_ids.shape
    idx_flat = token_ids.reshape(-1).astype(jnp.int32)
    out = _emb_kernel(embedding_mat, idx_flat)
    return out.reshape(*shape, DIM)


# speedup vs baseline: 1.8754x; 1.0377x over previous
import jax
import jax.numpy as jnp
from jax import lax
from jax.experimental import pallas as pl
from jax.experimental.pallas import tpu as pltpu
from jax.experimental.pallas import tpu_sc as plsc

NUM_CORES = 2
NUM_SUBCORES = 16
NUM_WORKERS = NUM_CORES * NUM_SUBCORES

B_TOTAL = 16384 * 50
PER_WORKER = B_TOTAL // NUM_WORKERS
CHUNK = 800
NCHUNK = PER_WORKER // CHUNK
NBUF = 2
DIM = 64


def _make_kernel():
    mesh = plsc.VectorSubcoreMesh(
        core_axis_name="c", subcore_axis_name="s")

    @pl.kernel(
        out_type=jax.ShapeDtypeStruct((B_TOTAL, DIM), jnp.float32),
        mesh=mesh,
        scratch_types=[
            pltpu.VMEM((CHUNK,), jnp.int32),
            pltpu.VMEM((CHUNK,), jnp.int32),
            pltpu.VMEM((CHUNK, DIM), jnp.float32),
            pltpu.VMEM((CHUNK, DIM), jnp.float32),
            pltpu.SemaphoreType.DMA,
            pltpu.SemaphoreType.DMA,
            pltpu.SemaphoreType.DMA,
            pltpu.SemaphoreType.DMA,
        ],
        compiler_params=pltpu.CompilerParams(use_tc_tiling_on_sc=False),
    )
    def emb_kernel(table_hbm, idx_hbm, out_hbm,
                   idx0, idx1, rows0, rows1, g0, g1, o0, o1):
        wid = lax.axis_index("s") * NUM_CORES + lax.axis_index("c")
        wbase = wid * PER_WORKER
        idxs, rows, gsem, osem = (idx0, idx1), (rows0, rows1), (g0, g1), (o0, o1)

        for b in range(NBUF):
            pltpu.sync_copy(idx_hbm.at[pl.ds(wbase + b * CHUNK, CHUNK)], idxs[b])
            pltpu.async_copy(table_hbm.at[idxs[b]], rows[b], gsem[b])

        @pl.loop(0, NCHUNK - NBUF, step=NBUF)
        def _steady(g):
            for b in range(NBUF):
                base = wbase + (g + b) * CHUNK
                pltpu.make_async_copy(table_hbm.at[idxs[b]], rows[b], gsem[b]).wait()
                pltpu.async_copy(rows[b], out_hbm.at[pl.ds(base, CHUNK)], osem[b])
                pltpu.sync_copy(
                    idx_hbm.at[pl.ds(base + NBUF * CHUNK, CHUNK)], idxs[b])
                pltpu.make_async_copy(
                    rows[b], out_hbm.at[pl.ds(base, CHUNK)], osem[b]).wait()
                pltpu.async_copy(table_hbm.at[idxs[b]], rows[b], gsem[b])

        for b in range(NBUF):
            base = wbase + (NCHUNK - NBUF + b) * CHUNK
            pltpu.make_async_copy(table_hbm.at[idxs[b]], rows[b], gsem[b]).wait()
            pltpu.async_copy(rows[b], out_hbm.at[pl.ds(base, CHUNK)], osem[b])
        for b in range(NBUF):
            base = wbase + (NCHUNK - NBUF + b) * CHUNK
            pltpu.make_async_copy(
                rows[b], out_hbm.at[pl.ds(base, CHUNK)], osem[b]).wait()

    return emb_kernel


_emb_kernel = _make_kernel()


@jax.jit
def kernel(token_ids, embedding_mat):
    shape = token_ids.shape
    idx_flat = token_ids.reshape(-1).astype(jnp.int32)
    out = _emb_kernel(embedding_mat, idx_flat)
    return out.reshape(*shape, DIM)


# speedup vs baseline: 1.8791x; 1.0020x over previous
import jax
import jax.numpy as jnp
from jax import lax
from jax.experimental import pallas as pl
from jax.experimental.pallas import tpu as pltpu
from jax.experimental.pallas import tpu_sc as plsc

NUM_CORES = 2
NUM_SUBCORES = 16
NUM_WORKERS = NUM_CORES * NUM_SUBCORES

B_TOTAL = 16384 * 50
PER_WORKER = B_TOTAL // NUM_WORKERS
CHUNK = 400
NCHUNK = PER_WORKER // CHUNK
NBUF = 4
DIM = 64


def _make_kernel():
    mesh = plsc.VectorSubcoreMesh(
        core_axis_name="c", subcore_axis_name="s")

    @pl.kernel(
        out_type=jax.ShapeDtypeStruct((B_TOTAL, DIM), jnp.float32),
        mesh=mesh,
        scratch_types=(
            [pltpu.VMEM((CHUNK,), jnp.int32) for _ in range(NBUF)]
            + [pltpu.VMEM((CHUNK, DIM), jnp.float32) for _ in range(NBUF)]
            + [pltpu.SemaphoreType.DMA for _ in range(2 * NBUF)]
        ),
        compiler_params=pltpu.CompilerParams(use_tc_tiling_on_sc=False),
    )
    def emb_kernel(table_hbm, idx_hbm, out_hbm, *bufs):
        idxs = bufs[:NBUF]
        rows = bufs[NBUF:2 * NBUF]
        gsem = bufs[2 * NBUF:3 * NBUF]
        osem = bufs[3 * NBUF:4 * NBUF]
        wid = lax.axis_index("s") * NUM_CORES + lax.axis_index("c")
        wbase = wid * PER_WORKER

        for b in range(NBUF):
            pltpu.sync_copy(idx_hbm.at[pl.ds(wbase + b * CHUNK, CHUNK)], idxs[b])
            pltpu.async_copy(table_hbm.at[idxs[b]], rows[b], gsem[b])

        @pl.loop(0, NCHUNK - NBUF, step=NBUF)
        def _steady(g):
            for b in range(NBUF):
                base = wbase + (g + b) * CHUNK
                pltpu.make_async_copy(table_hbm.at[idxs[b]], rows[b], gsem[b]).wait()
                pltpu.async_copy(rows[b], out_hbm.at[pl.ds(base, CHUNK)], osem[b])
                pltpu.sync_copy(
                    idx_hbm.at[pl.ds(base + NBUF * CHUNK, CHUNK)], idxs[b])
                pltpu.make_async_copy(
                    rows[b], out_hbm.at[pl.ds(base, CHUNK)], osem[b]).wait()
                pltpu.async_copy(table_hbm.at[idxs[b]], rows[b], gsem[b])

        for b in range(NBUF):
            base = wbase + (NCHUNK - NBUF + b) * CHUNK
            pltpu.make_async_copy(table_hbm.at[idxs[b]], rows[b], gsem[b]).wait()
            pltpu.async_copy(rows[b], out_hbm.at[pl.ds(base, CHUNK)], osem[b])
        for b in range(NBUF):
            base = wbase + (NCHUNK - NBUF + b) * CHUNK
            pltpu.make_async_copy(
                rows[b], out_hbm.at[pl.ds(base, CHUNK)], osem[b]).wait()

    return emb_kernel


_emb_kernel = _make_kernel()


@jax.jit
def kernel(token_ids, embedding_mat):
    shape = token_ids.shape
    idx_flat = token_ids.reshape(-1).astype(jnp.int32)
    out = _emb_kernel(embedding_mat, idx_flat)
    return out.reshape(*shape, DIM)
